# baseline (speedup 1.0000x reference)
import jax
import jax.numpy as jnp
from jax.experimental import pallas as pl
from jax.experimental.pallas import tpu as pltpu
from jax.experimental.pallas import tpu_sc as plsc

BATCH = 4096
SEQ_LEN = 200
EMBEDDING_DIM = 64
NUM_IDX = BATCH * SEQ_LEN
WINDOW = 128


def _sc_gather(weight, flat_idx):
    mesh = plsc.VectorSubcoreMesh(core_axis_name="core", subcore_axis_name="subcore")

    @pl.kernel(
        out_type=jax.ShapeDtypeStruct((NUM_IDX, EMBEDDING_DIM), weight.dtype),
        mesh=mesh,
        compiler_params=pltpu.CompilerParams(use_tc_tiling_on_sc=False),
    )
    def k(w_hbm, i_hbm, o_hbm):
        def body(i_vmem, o_vmem):
            pltpu.sync_copy(w_hbm.at[i_vmem.at[0]], o_vmem)

        pltpu.emit_pipeline(
            body,
            grid=(NUM_IDX // WINDOW,),
            in_specs=[pl.BlockSpec((1, WINDOW), index_map=lambda i: (0, i))],
            out_specs=[
                pl.BlockSpec((WINDOW, EMBEDDING_DIM), index_map=lambda i: (i, 0))
            ],
            core_axis_name=("core", "subcore"),
            dimension_semantics=(pltpu.PARALLEL,),
        )(i_hbm, o_hbm)

    return k(weight, flat_idx)


def kernel(token_ids, weight):
    flat_idx = token_ids.reshape(1, NUM_IDX)
    out = _sc_gather(weight, flat_idx)
    return out.reshape(BATCH, SEQ_LEN, EMBEDDING_DIM)


# speedup vs baseline: 1.0714x; 1.0714x over previous
import jax
import jax.numpy as jnp
from jax.experimental import pallas as pl
from jax.experimental.pallas import tpu as pltpu
from jax.experimental.pallas import tpu_sc as plsc

BATCH = 4096
SEQ_LEN = 200
EMBEDDING_DIM = 64
NUM_IDX = BATCH * SEQ_LEN
WINDOW = 512
SUB = 128


def _sc_gather(weight, flat_idx):
    mesh = plsc.VectorSubcoreMesh(core_axis_name="core", subcore_axis_name="subcore")

    @pl.kernel(
        out_type=jax.ShapeDtypeStruct((NUM_IDX, EMBEDDING_DIM), weight.dtype),
        mesh=mesh,
        scratch_types=[pltpu.SemaphoreType.DMA],
        compiler_params=pltpu.CompilerParams(use_tc_tiling_on_sc=False),
    )
    def k(w_hbm, i_hbm, o_hbm, sem):
        def body(i_vmem, o_vmem):
            copies = [
                pltpu.async_copy(
                    w_hbm.at[i_vmem.at[0, pl.ds(j * SUB, SUB)]],
                    o_vmem.at[pl.ds(j * SUB, SUB)],
                    sem,
                )
                for j in range(WINDOW // SUB)
            ]
            for c in copies:
                c.wait()

        pltpu.emit_pipeline(
            body,
            grid=(NUM_IDX // WINDOW,),
            in_specs=[pl.BlockSpec((1, WINDOW), index_map=lambda i: (0, i))],
            out_specs=[
                pl.BlockSpec((WINDOW, EMBEDDING_DIM), index_map=lambda i: (i, 0))
            ],
            core_axis_name=("core", "subcore"),
            dimension_semantics=(pltpu.PARALLEL,),
        )(i_hbm, o_hbm)

    return k(weight, flat_idx)


def kernel(token_ids, weight):
    flat_idx = token_ids.reshape(1, NUM_IDX)
    out = _sc_gather(weight, flat_idx)
    return out.reshape(BATCH, SEQ_LEN, EMBEDDING_DIM)


# speedup vs baseline: 1.3091x; 1.2218x over previous
import jax
import jax.numpy as jnp
from jax.experimental import pallas as pl
from jax.experimental.pallas import tpu as pltpu
from jax.experimental.pallas import tpu_sc as plsc

BATCH = 4096
SEQ_LEN = 200
EMBEDDING_DIM = 64
PADDED_DIM = 128
NUM_IDX = BATCH * SEQ_LEN
WINDOW = 256
SUB = 128


def _sc_gather(w128, flat_idx):
    mesh = plsc.VectorSubcoreMesh(core_axis_name="core", subcore_axis_name="subcore")

    @pl.kernel(
        out_type=jax.ShapeDtypeStruct((NUM_IDX, PADDED_DIM), w128.dtype),
        mesh=mesh,
        scratch_types=[pltpu.SemaphoreType.DMA],
    )
    def k(w_hbm, i_hbm, o_hbm, sem):
        def body(i_vmem, o_vmem):
            copies = [
                pltpu.async_copy(
                    w_hbm.at[i_vmem.at[0, pl.ds(j * SUB, SUB)]],
                    o_vmem.at[pl.ds(j * SUB, SUB)],
                    sem,
                )
                for j in range(WINDOW // SUB)
            ]
            for c in copies:
                c.wait()

        pltpu.emit_pipeline(
            body,
            grid=(NUM_IDX // WINDOW,),
            in_specs=[pl.BlockSpec((1, WINDOW), index_map=lambda i: (0, i))],
            out_specs=[pl.BlockSpec((WINDOW, PADDED_DIM), index_map=lambda i: (i, 0))],
            core_axis_name=("core", "subcore"),
            dimension_semantics=(pltpu.PARALLEL,),
        )(i_hbm, o_hbm)

    return k(w128, flat_idx)


def kernel(token_ids, weight):
    w128 = jnp.pad(weight, ((0, 0), (0, PADDED_DIM - EMBEDDING_DIM)))
    flat_idx = token_ids.reshape(1, NUM_IDX)
    out = _sc_gather(w128, flat_idx)
    return out[:, :EMBEDDING_DIM].reshape(BATCH, SEQ_LEN, EMBEDDING_DIM)


# speedup vs baseline: 1.3108x; 1.0013x over previous
import jax
import jax.numpy as jnp
from jax import lax
from jax.experimental import pallas as pl
from jax.experimental.pallas import tpu as pltpu
from jax.experimental.pallas import tpu_sc as plsc

BATCH = 4096
SEQ_LEN = 200
EMBEDDING_DIM = 64
PADDED_DIM = 128
NUM_IDX = BATCH * SEQ_LEN

NUM_WORKERS = 32
PER_W = NUM_IDX // NUM_WORKERS
WIN = 128
NWIN = PER_W // WIN
NBUF = 5
LOOK = 2


def _sc_gather(w128, flat_idx):
    mesh = plsc.VectorSubcoreMesh(core_axis_name="core", subcore_axis_name="subcore")

    @pl.kernel(
        out_type=jax.ShapeDtypeStruct((NUM_IDX, PADDED_DIM), w128.dtype),
        mesh=mesh,
        scratch_types=[
            pltpu.VMEM((PER_W,), jnp.int32),
            pltpu.VMEM((NBUF, WIN, PADDED_DIM), jnp.float32),
            pltpu.SemaphoreType.DMA((NBUF,)),
            pltpu.SemaphoreType.DMA((NBUF,)),
        ],
    )
    def k(w_hbm, i_hbm, o_hbm, idx_v, bufs, gsem, osem):
        wid = lax.axis_index("subcore") * 2 + lax.axis_index("core")
        base = wid * PER_W
        pltpu.sync_copy(i_hbm.at[pl.ds(base, PER_W)], idx_v)

        def gstart(w, slot):
            pltpu.async_copy(
                w_hbm.at[idx_v.at[pl.ds(w * WIN, WIN)]],
                bufs.at[slot],
                gsem.at[slot],
            )

        def gwait(slot):
            pltpu.make_async_copy(
                o_hbm.at[pl.ds(0, WIN)], bufs.at[slot], gsem.at[slot]
            ).wait()

        def ostart(w, slot):
            pltpu.async_copy(
                bufs.at[slot],
                o_hbm.at[pl.ds(base + w * WIN, WIN)],
                osem.at[slot],
            )

        def owait(slot):
            pltpu.make_async_copy(
                bufs.at[slot], o_hbm.at[pl.ds(0, WIN)], osem.at[slot]
            ).wait()

        def visit(w, slot, *, head_skip_owait=False, tail_skip_gstart=False):
            gwait(slot)
            ostart(w, slot)
            if not tail_skip_gstart:
                if not head_skip_owait:
                    owait((slot + LOOK) % NBUF)
                gstart(w + LOOK, (slot + LOOK) % NBUF)

        for w in range(LOOK):
            gstart(w, w % NBUF)
        for w in range(NBUF):
            visit(w, w % NBUF, head_skip_owait=(w < NBUF - LOOK))

        @pl.loop(NBUF, NWIN - NBUF, step=NBUF)
        def _(wbase):
            for j in range(NBUF):
                visit(wbase + j, j)

        for w in range(NWIN - NBUF, NWIN):
            visit(w, w % NBUF, tail_skip_gstart=(w + LOOK >= NWIN))
        for w in range(NWIN - NBUF, NWIN):
            owait(w % NBUF)

    return k(w128, flat_idx)


def kernel(token_ids, weight):
    w128 = jnp.pad(weight, ((0, 0), (0, PADDED_DIM - EMBEDDING_DIM)))
    flat_idx = token_ids.reshape(NUM_IDX)
    out = _sc_gather(w128, flat_idx)
    return out[:, :EMBEDDING_DIM].reshape(BATCH, SEQ_LEN, EMBEDDING_DIM)


# speedup vs baseline: 1.4304x; 1.0913x over previous
import jax
import jax.numpy as jnp
from jax import lax
from jax.experimental import pallas as pl
from jax.experimental.pallas import tpu as pltpu
from jax.experimental.pallas import tpu_sc as plsc

BATCH = 4096
SEQ_LEN = 200
EMBEDDING_DIM = 64
PADDED_DIM = 128
NUM_IDX = BATCH * SEQ_LEN

NUM_WORKERS = 32
PER_W = NUM_IDX // NUM_WORKERS
WIN = 128
NWIN = PER_W // WIN
NBUF = 10
LOOK = 5


def _sc_gather(weight, flat_idx):
    mesh = plsc.VectorSubcoreMesh(core_axis_name="core", subcore_axis_name="subcore")

    @pl.kernel(
        out_type=jax.ShapeDtypeStruct((NUM_IDX, PADDED_DIM), weight.dtype),
        mesh=mesh,
        compiler_params=pltpu.CompilerParams(use_tc_tiling_on_sc=False),
        scratch_types=[
            pltpu.VMEM((PER_W,), jnp.int32),
            pltpu.VMEM((NBUF, WIN, EMBEDDING_DIM), jnp.float32),
            pltpu.SemaphoreType.DMA((NBUF,)),
            pltpu.SemaphoreType.DMA((NBUF,)),
        ],
    )
    def k(w_hbm, i_hbm, o_hbm, idx_v, bufs, gsem, osem):
        wid = lax.axis_index("subcore") * 2 + lax.axis_index("core")
        base = wid * PER_W
        pltpu.sync_copy(i_hbm.at[pl.ds(base, PER_W)], idx_v)

        def gstart(w, slot):
            pltpu.async_copy(
                w_hbm.at[idx_v.at[pl.ds(w * WIN, WIN)]],
                bufs.at[slot],
                gsem.at[slot],
            )

        def gwait(slot):
            pltpu.make_async_copy(
                o_hbm.at[pl.ds(0, WIN), pl.ds(0, EMBEDDING_DIM)],
                bufs.at[slot],
                gsem.at[slot],
            ).wait()

        def ostart(w, slot):
            pltpu.async_copy(
                bufs.at[slot],
                o_hbm.at[pl.ds(base + w * WIN, WIN), pl.ds(0, EMBEDDING_DIM)],
                osem.at[slot],
            )

        def owait(slot):
            pltpu.make_async_copy(
                bufs.at[slot],
                o_hbm.at[pl.ds(0, WIN), pl.ds(0, EMBEDDING_DIM)],
                osem.at[slot],
            ).wait()

        def visit(w, slot, *, head_skip_owait=False, tail_skip_gstart=False):
            gwait(slot)
            ostart(w, slot)
            if not tail_skip_gstart:
                if not head_skip_owait:
                    owait((slot + LOOK) % NBUF)
                gstart(w + LOOK, (slot + LOOK) % NBUF)

        for w in range(LOOK):
            gstart(w, w % NBUF)
        for w in range(NBUF):
            visit(w, w % NBUF, head_skip_owait=(w < NBUF - LOOK))

        @pl.loop(NBUF, NWIN - NBUF, step=NBUF)
        def _(wbase):
            for j in range(NBUF):
                visit(wbase + j, j)

        for w in range(NWIN - NBUF, NWIN):
            visit(w, w % NBUF, tail_skip_gstart=(w + LOOK >= NWIN))
        for w in range(NWIN - NBUF, NWIN):
            owait(w % NBUF)

    return k(weight, flat_idx)


def kernel(token_ids, weight):
    flat_idx = token_ids.reshape(NUM_IDX)
    out = _sc_gather(weight, flat_idx)
    return out[:, :EMBEDDING_DIM].reshape(BATCH, SEQ_LEN, EMBEDDING_DIM)
